# baseline (speedup 1.0000x reference)
import functools

import jax
import jax.numpy as jnp
from jax import lax
from jax.experimental import pallas as pl
from jax.experimental.pallas import tpu as pltpu
from jax.experimental.pallas import tpu_sc as plsc

D_MODEL = 64
SCALE = 8.0

CB = 2
SEG = (104, 96)
NBUF = 2


@functools.lru_cache(maxsize=None)
def _make_gather(n_b0: int, n_b1: int):
  info = plsc.get_sparse_core_info()
  nc, ns = info.num_cores, info.num_subcores
  nw = nc * ns
  rows_per_w = n_b0 // nw
  chunks = rows_per_w // CB
  mesh = plsc.VectorSubcoreMesh(core_axis_name="c", subcore_axis_name="s")

  @functools.partial(
      pl.kernel,
      mesh=mesh,
      compiler_params=pltpu.CompilerParams(use_tc_tiling_on_sc=False),
      out_type=jax.ShapeDtypeStruct((n_b0, n_b1, D_MODEL), jnp.float32),
      scratch_types=[
          pltpu.VMEM((NBUF, CB, n_b1), jnp.int32),
          pltpu.VMEM((NBUF, CB, n_b1, D_MODEL), jnp.float32),
          pltpu.SemaphoreType.DMA,
          pltpu.SemaphoreType.DMA,
          pltpu.SemaphoreType.DMA,
          pltpu.SemaphoreType.DMA,
      ],
  )
  def k(x_hbm, table_hbm, out_hbm, idx_v, rows_v, sg0, sg1, sw0, sw1):
    wid = lax.axis_index("s") * nc + lax.axis_index("c")
    b0w = wid * rows_per_w
    sem_g = (sg0, sg1)
    sem_w = (sw0, sw1)

    def gather_copies(b):
      cps = []
      for cb in range(CB):
        off = 0
        for ln in SEG:
          cps.append(pltpu.make_async_copy(
              table_hbm.at[idx_v.at[b, cb, pl.ds(off, ln)]],
              rows_v.at[b, cb, pl.ds(off, ln)],
              sem_g[b]))
          off += ln
      return cps

    def fire(c, b):
      pltpu.sync_copy(x_hbm.at[pl.ds(b0w + c * CB, CB)], idx_v.at[b])
      for cp in gather_copies(b):
        cp.start()

    def drain_gathers(b):
      for cp in gather_copies(b):
        cp.wait()

    def out_copy(c, b):
      return pltpu.make_async_copy(
          rows_v.at[b], out_hbm.at[pl.ds(b0w + c * CB, CB)], sem_w[b])

    def put(c, b):
      for cb in range(CB):
        @pl.loop(0, n_b1, unroll=8)
        def _(t):
          for kk in range(D_MODEL // 16):
            sl = pl.ds(kk * 16, 16)
            rows_v[b, cb, t, sl] = rows_v[b, cb, t, sl] * SCALE

      out_copy(c, b).start()

    fire(0, 0)

    @pl.loop(0, chunks, step=NBUF)
    def _(c0):
      for boff in range(NBUF):
        c = c0 + boff
        b = boff
        nb = 1 - boff

        @pl.when(c >= 1)
        def _():
          out_copy(c - 1, nb).wait()

        @pl.when(c + 1 < chunks)
        def _():
          fire(c + 1, nb)

        drain_gathers(b)
        put(c, b)

    out_copy(chunks - 1, (chunks - 1) % NBUF).wait()

  return k


def kernel(x, table):
  b0, b1 = x.shape
  return _make_gather(b0, b1)(x.astype(jnp.int32), table)
